# baseline (speedup 1.0000x reference)
import functools
import math

import jax
import jax.numpy as jnp
from jax import lax
from jax.experimental import pallas as pl
from jax.experimental.pallas import tpu as pltpu
from jax.experimental.pallas import tpu_sc as plsc

DATA_LEN = 100000
RHO = 0.001
LR_DUAL = 0.001
MARGIN = 1.0
LAM = 1.0
LOG_RHO = math.log(RHO)

B = 4096
NC, NS = 2, 16
NW = NC * NS
GATHER_PER_W = B // NW
CHUNK = 3136
PAD_LEN = NW * CHUNK

R_BLK = 256
NB = B // R_BLK


def _sc_gather_body(alpha_hbm, idx_hbm, out_hbm, idx_v, rows_v, sem):
    wid = lax.axis_index("s") * NC + lax.axis_index("c")
    base = wid * GATHER_PER_W
    pltpu.sync_copy(idx_hbm.at[pl.ds(base, GATHER_PER_W)], idx_v)
    pltpu.async_copy(alpha_hbm.at[idx_v], rows_v, sem).wait()
    pltpu.sync_copy(rows_v, out_hbm.at[pl.ds(base, GATHER_PER_W)])


def _sc_gather(alpha_flat, index):
    mesh = plsc.VectorSubcoreMesh(
        core_axis_name="c", subcore_axis_name="s", num_cores=NC, num_subcores=NS)
    return pl.kernel(
        _sc_gather_body,
        out_type=jax.ShapeDtypeStruct((B,), jnp.float32),
        mesh=mesh,
        scratch_types=[
            pltpu.VMEM((GATHER_PER_W,), jnp.int32),
            pltpu.VMEM((GATHER_PER_W,), jnp.float32),
            pltpu.SemaphoreType.DMA,
        ],
    )(alpha_flat, index)


def _tc_dense_body(p_col, yt_col, a_col, i_col, p_row, yt_row, i_row,
                   a_new_ref, win_ref, loss_ref, acc):
    r = pl.program_id(0)
    pr = p_col[...]
    ar = a_col[...]
    posr = (yt_col[...] == 1.0)
    ir = i_col[...]
    pc = p_row[...]
    ytc = yt_row[...]
    ic = i_row[...]
    negc = (ytc == 0.0).astype(jnp.float32)
    num_neg = jnp.sum(negc)
    num_pos = jnp.sum((ytc == 1.0).astype(jnp.float32))

    t = MARGIN - pr + pc
    h = jnp.maximum(t, 0.0)
    e = h * h - ar + LOG_RHO
    m = jnp.maximum(e, 0.0)
    u = jnp.exp(jnp.minimum(e, -e))
    sp = m + jnp.log1p(u)
    sg = jnp.where(e >= 0.0, 1.0, u) / (1.0 + u)
    S = jnp.sum(sp * negc, axis=1, keepdims=True)
    W = jnp.sum(sg * negc, axis=1, keepdims=True)
    a_new_ref[...] = ar - LR_DUAL * (1.0 - W / num_neg)

    colj = lax.broadcasted_iota(jnp.int32, (R_BLK, B), 1)
    rowi = lax.broadcasted_iota(jnp.int32, (R_BLK, B), 0) + r * R_BLK
    dup = ((ic == ir) & (ytc == 1.0) & (colj > rowi)).astype(jnp.float32)
    lose = jnp.sum(dup, axis=1, keepdims=True) > 0.0
    win_ref[...] = jnp.where(posr & (~lose), 1.0, 0.0)

    posr_f = posr.astype(jnp.float32)
    part_S = jnp.sum(posr_f * S)
    part_a = jnp.sum(posr_f * ar)

    @pl.when(r == 0)
    def _init():
        acc[0] = part_S
        acc[1] = part_a

    @pl.when(r > 0)
    def _accum():
        acc[0] = acc[0] + part_S
        acc[1] = acc[1] + part_a

    @pl.when(r == NB - 1)
    def _final():
        val = (LAM / RHO) * acc[0] / (num_pos * num_neg) + acc[1] / num_pos
        loss_ref[...] = jnp.reshape(val, (1, 1))


def _tc_dense(p_col, yt_col, a_col, i_col, p_row, yt_row, i_row):
    col_spec = lambda dt: pl.BlockSpec((R_BLK, 1), lambda r: (r, 0))
    row_spec = lambda dt: pl.BlockSpec((1, B), lambda r: (0, 0))
    return pl.pallas_call(
        _tc_dense_body,
        grid=(NB,),
        in_specs=[col_spec(None), col_spec(None), col_spec(None), col_spec(None),
                  row_spec(None), row_spec(None), row_spec(None)],
        out_specs=[pl.BlockSpec((R_BLK, 1), lambda r: (r, 0)),
                   pl.BlockSpec((R_BLK, 1), lambda r: (r, 0)),
                   pl.BlockSpec((1, 1), lambda r: (0, 0))],
        out_shape=[jax.ShapeDtypeStruct((B, 1), jnp.float32),
                   jax.ShapeDtypeStruct((B, 1), jnp.float32),
                   jax.ShapeDtypeStruct((1, 1), jnp.float32)],
        scratch_shapes=[pltpu.SMEM((2,), jnp.float32)],
    )(p_col, yt_col, a_col, i_col, p_row, yt_row, i_row)


def _sc_scatter_body(alpha_hbm, idx_hbm, val_hbm, win_hbm, out_hbm,
                     tbl_v, idx_v, val_v, win_v):
    wid = lax.axis_index("s") * NC + lax.axis_index("c")
    lo = wid * CHUNK
    pltpu.sync_copy(alpha_hbm.at[pl.ds(lo, CHUNK)], tbl_v)
    pltpu.sync_copy(idx_hbm, idx_v)
    pltpu.sync_copy(val_hbm, val_v)
    pltpu.sync_copy(win_hbm, win_v)

    def body(k, carry):
        off = pl.multiple_of(k * 16, 16)
        iv = idx_v[pl.ds(off, 16)]
        vv = val_v[pl.ds(off, 16)]
        wv = win_v[pl.ds(off, 16)]
        msk = (iv >= lo) & (iv < lo + CHUNK) & (wv > 0.0)
        plsc.store_scatter(tbl_v, [iv - lo], vv, mask=msk)
        return carry

    lax.fori_loop(0, B // 16, body, 0, unroll=8)
    pltpu.sync_copy(tbl_v, out_hbm.at[pl.ds(lo, CHUNK)])


def _sc_scatter(alpha_pad, index, vals, win):
    mesh = plsc.VectorSubcoreMesh(
        core_axis_name="c", subcore_axis_name="s", num_cores=NC, num_subcores=NS)
    return pl.kernel(
        _sc_scatter_body,
        out_type=jax.ShapeDtypeStruct((PAD_LEN,), jnp.float32),
        mesh=mesh,
        scratch_types=[
            pltpu.VMEM((CHUNK,), jnp.float32),
            pltpu.VMEM((B,), jnp.int32),
            pltpu.VMEM((B,), jnp.float32),
            pltpu.VMEM((B,), jnp.float32),
        ],
        compiler_params=pltpu.CompilerParams(needs_layout_passes=False),
    )(alpha_pad, index, vals, win)


def kernel(y_pred, y_true, index, alpha):
    p_col = y_pred.reshape(B, 1)
    yt_col = y_true.reshape(B, 1)
    index = index.reshape(B).astype(jnp.int32)
    i_col = index.reshape(B, 1)
    alpha_flat = alpha.reshape(DATA_LEN)
    alpha_pad = jnp.pad(alpha_flat, (0, PAD_LEN - DATA_LEN))

    a = _sc_gather(alpha_pad, index)

    a_new, win, loss = _tc_dense(
        p_col, yt_col, a.reshape(B, 1), i_col,
        p_col.reshape(1, B), yt_col.reshape(1, B), i_col.reshape(1, B))

    alpha_out = _sc_scatter(alpha_pad, index, a_new.reshape(B), win.reshape(B))
    return loss.reshape(()), alpha_out[:DATA_LEN].reshape(DATA_LEN, 1)


# speedup vs baseline: 1.0242x; 1.0242x over previous
import functools
import math

import jax
import jax.numpy as jnp
from jax import lax
from jax.experimental import pallas as pl
from jax.experimental.pallas import tpu as pltpu
from jax.experimental.pallas import tpu_sc as plsc

DATA_LEN = 100000
RHO = 0.001
LR_DUAL = 0.001
MARGIN = 1.0
LAM = 1.0
LOG_RHO = math.log(RHO)

B = 4096
NC, NS = 2, 16
NW = NC * NS
GATHER_PER_W = B // NW
CHUNK = 3136
PAD_LEN = NW * CHUNK

R_BLK = 256
NB = B // R_BLK


def _sc_gather_body(alpha_hbm, idx_hbm, out_hbm, idx_v, rows_v, sem):
    wid = lax.axis_index("s") * NC + lax.axis_index("c")
    base = wid * GATHER_PER_W
    pltpu.sync_copy(idx_hbm.at[pl.ds(base, GATHER_PER_W)], idx_v)
    pltpu.async_copy(alpha_hbm.at[idx_v], rows_v, sem).wait()
    pltpu.sync_copy(rows_v, out_hbm.at[pl.ds(base, GATHER_PER_W)])


def _sc_gather(alpha_flat, index):
    mesh = plsc.VectorSubcoreMesh(
        core_axis_name="c", subcore_axis_name="s", num_cores=NC, num_subcores=NS)
    return pl.kernel(
        _sc_gather_body,
        out_type=jax.ShapeDtypeStruct((B,), jnp.float32),
        mesh=mesh,
        scratch_types=[
            pltpu.VMEM((GATHER_PER_W,), jnp.int32),
            pltpu.VMEM((GATHER_PER_W,), jnp.float32),
            pltpu.SemaphoreType.DMA,
        ],
    )(alpha_flat, index)


def _tc_dense_body(p_col, yt_col, a_col, i_col, p_row, yt_row, i_row,
                   a_new_ref, win_ref, loss_ref, acc):
    r = pl.program_id(0)
    pr = p_col[...]
    ar = a_col[...]
    posr = (yt_col[...] == 1.0)
    ir = i_col[...]
    pc = p_row[...]
    ytc = yt_row[...]
    ic = i_row[...]
    negm = (ytc == 0.0)
    num_neg = jnp.sum(negm.astype(jnp.float32))
    num_pos = jnp.sum((ytc == 1.0).astype(jnp.float32))

    cr = LOG_RHO - ar
    t = pc - pr
    h = jnp.maximum(t, 0.0)
    e = jnp.where(negm, h * h + cr, -jnp.inf)
    m = jnp.maximum(e, 0.0)
    u = jnp.exp(jnp.minimum(e, -e))
    sp = m + jnp.log1p(u)
    sg = 0.5 * jnp.tanh(0.5 * e) + 0.5
    S = jnp.sum(sp, axis=1, keepdims=True)
    W = jnp.sum(sg, axis=1, keepdims=True)
    a_new_ref[...] = ar - LR_DUAL * (1.0 - W / num_neg)

    colj = lax.broadcasted_iota(jnp.int32, (R_BLK, B), 1)
    rowi = lax.broadcasted_iota(jnp.int32, (R_BLK, B), 0) + r * R_BLK
    dup = ((ic == ir) & (ytc == 1.0) & (colj > rowi)).astype(jnp.float32)
    lose = jnp.sum(dup, axis=1, keepdims=True) > 0.0
    win_ref[...] = jnp.where(posr & (~lose), 1.0, 0.0)

    posr_f = posr.astype(jnp.float32)
    part_S = jnp.sum(posr_f * S)
    part_a = jnp.sum(posr_f * ar)

    @pl.when(r == 0)
    def _init():
        acc[0] = part_S
        acc[1] = part_a

    @pl.when(r > 0)
    def _accum():
        acc[0] = acc[0] + part_S
        acc[1] = acc[1] + part_a

    @pl.when(r == NB - 1)
    def _final():
        val = (LAM / RHO) * acc[0] / (num_pos * num_neg) + acc[1] / num_pos
        loss_ref[...] = jnp.reshape(val, (1, 1))


def _tc_dense(p_col, yt_col, a_col, i_col, p_row, yt_row, i_row):
    col_spec = lambda dt: pl.BlockSpec((R_BLK, 1), lambda r: (r, 0))
    row_spec = lambda dt: pl.BlockSpec((1, B), lambda r: (0, 0))
    return pl.pallas_call(
        _tc_dense_body,
        grid=(NB,),
        in_specs=[col_spec(None), col_spec(None), col_spec(None), col_spec(None),
                  row_spec(None), row_spec(None), row_spec(None)],
        out_specs=[pl.BlockSpec((R_BLK, 1), lambda r: (r, 0)),
                   pl.BlockSpec((R_BLK, 1), lambda r: (r, 0)),
                   pl.BlockSpec((1, 1), lambda r: (0, 0))],
        out_shape=[jax.ShapeDtypeStruct((B, 1), jnp.float32),
                   jax.ShapeDtypeStruct((B, 1), jnp.float32),
                   jax.ShapeDtypeStruct((1, 1), jnp.float32)],
        scratch_shapes=[pltpu.SMEM((2,), jnp.float32)],
    )(p_col, yt_col, a_col, i_col, p_row, yt_row, i_row)


def _sc_scatter_body(alpha_hbm, idx_hbm, val_hbm, win_hbm, out_hbm,
                     tbl_v, idx_v, val_v, win_v):
    wid = lax.axis_index("s") * NC + lax.axis_index("c")
    lo = wid * CHUNK
    pltpu.sync_copy(alpha_hbm.at[pl.ds(lo, CHUNK)], tbl_v)
    pltpu.sync_copy(idx_hbm, idx_v)
    pltpu.sync_copy(val_hbm, val_v)
    pltpu.sync_copy(win_hbm, win_v)

    def body(k, carry):
        off = pl.multiple_of(k * 16, 16)
        iv = idx_v[pl.ds(off, 16)]
        vv = val_v[pl.ds(off, 16)]
        wv = win_v[pl.ds(off, 16)]
        msk = (iv >= lo) & (iv < lo + CHUNK) & (wv > 0.0)
        plsc.store_scatter(tbl_v, [iv - lo], vv, mask=msk)
        return carry

    lax.fori_loop(0, B // 16, body, 0, unroll=8)
    pltpu.sync_copy(tbl_v, out_hbm.at[pl.ds(lo, CHUNK)])


def _sc_scatter(alpha_pad, index, vals, win):
    mesh = plsc.VectorSubcoreMesh(
        core_axis_name="c", subcore_axis_name="s", num_cores=NC, num_subcores=NS)
    return pl.kernel(
        _sc_scatter_body,
        out_type=jax.ShapeDtypeStruct((PAD_LEN,), jnp.float32),
        mesh=mesh,
        scratch_types=[
            pltpu.VMEM((CHUNK,), jnp.float32),
            pltpu.VMEM((B,), jnp.int32),
            pltpu.VMEM((B,), jnp.float32),
            pltpu.VMEM((B,), jnp.float32),
        ],
        compiler_params=pltpu.CompilerParams(needs_layout_passes=False),
    )(alpha_pad, index, vals, win)


def kernel(y_pred, y_true, index, alpha):
    p_col = y_pred.reshape(B, 1)
    yt_col = y_true.reshape(B, 1)
    index = index.reshape(B).astype(jnp.int32)
    i_col = index.reshape(B, 1)
    alpha_flat = alpha.reshape(DATA_LEN)
    alpha_pad = jnp.pad(alpha_flat, (0, PAD_LEN - DATA_LEN))

    a = _sc_gather(alpha_pad, index)

    a_new, win, loss = _tc_dense(
        p_col, yt_col, a.reshape(B, 1), i_col,
        (p_col + MARGIN).reshape(1, B), yt_col.reshape(1, B), i_col.reshape(1, B))

    alpha_out = _sc_scatter(alpha_pad, index, a_new.reshape(B), win.reshape(B))
    return loss.reshape(()), alpha_out[:DATA_LEN].reshape(DATA_LEN, 1)


# speedup vs baseline: 1.2522x; 1.2226x over previous
import functools
import math

import jax
import jax.numpy as jnp
from jax import lax
from jax.experimental import pallas as pl
from jax.experimental.pallas import tpu as pltpu
from jax.experimental.pallas import tpu_sc as plsc

DATA_LEN = 100000
RHO = 0.001
LR_DUAL = 0.001
MARGIN = 1.0
LAM = 1.0
LOG_RHO = math.log(RHO)

B = 4096
NC, NS = 2, 16
NW = NC * NS
GATHER_PER_W = B // NW
CHUNK = 3136
PAD_LEN = NW * CHUNK

R_BLK = 256
NB = B // R_BLK


def _sc_gather_body(alpha_hbm, idx_hbm, out_hbm, idx_v, rows_v, sem):
    wid = lax.axis_index("s") * NC + lax.axis_index("c")
    base = wid * GATHER_PER_W
    pltpu.sync_copy(idx_hbm.at[pl.ds(base, GATHER_PER_W)], idx_v)
    pltpu.async_copy(alpha_hbm.at[idx_v], rows_v, sem).wait()
    pltpu.sync_copy(rows_v, out_hbm.at[pl.ds(base, GATHER_PER_W)])


def _sc_gather(alpha_flat, index):
    mesh = plsc.VectorSubcoreMesh(
        core_axis_name="c", subcore_axis_name="s", num_cores=NC, num_subcores=NS)
    return pl.kernel(
        _sc_gather_body,
        out_type=jax.ShapeDtypeStruct((B,), jnp.float32),
        mesh=mesh,
        scratch_types=[
            pltpu.VMEM((GATHER_PER_W,), jnp.int32),
            pltpu.VMEM((GATHER_PER_W,), jnp.float32),
            pltpu.SemaphoreType.DMA,
        ],
    )(alpha_flat, index)


LOG2E = 1.4426950408889634
LN2 = 0.6931471805599453
SQRT_LOG2E = LOG2E ** 0.5


def _tc_dense_body(p_col, yt_col, a_col, k_col, p_row, yt_row, k_row,
                   a_new_ref, win_ref, loss_ref, acc):
    r = pl.program_id(0)
    pr = p_col[...]
    ar = a_col[...]
    posr = (yt_col[...] == 1.0)
    kr = k_col[...]
    pc = p_row[...]
    ytc = yt_row[...]
    kc = k_row[...]
    negm = (ytc == 0.0)
    num_neg = jnp.sum(negm.astype(jnp.float32))
    num_pos = jnp.sum((ytc == 1.0).astype(jnp.float32))

    cr2 = (LOG_RHO - ar) * LOG2E
    h = jnp.maximum(pc - pr, 0.0)
    e2 = jnp.where(negm, h * h + cr2, -jnp.inf)
    nabs = lax.bitcast_convert_type(
        lax.bitcast_convert_type(e2, jnp.int32) | jnp.int32(-2147483648),
        jnp.float32)
    u = jnp.exp2(nabs)
    l2 = jnp.log2(1.0 + u)
    m2 = jnp.maximum(e2, 0.0)
    th = jnp.tanh(e2 * (LN2 * 0.5))
    Sm = jnp.sum(m2, axis=1, keepdims=True)
    Sl = jnp.sum(l2, axis=1, keepdims=True)
    Th = jnp.sum(th, axis=1, keepdims=True)
    S = LN2 * (Sm + Sl)
    W = 0.5 * (B + Th)
    a_new_ref[...] = ar - LR_DUAL * (1.0 - W / num_neg)

    delta = lax.bitcast_convert_type(kc - kr - 1, jnp.uint32)
    lose = jnp.any(delta < jnp.uint32(4095), axis=1, keepdims=True)
    win_ref[...] = jnp.where(posr & (~lose), 1.0, 0.0)

    posr_f = posr.astype(jnp.float32)
    part_S = jnp.sum(posr_f * S)
    part_a = jnp.sum(posr_f * ar)

    @pl.when(r == 0)
    def _init():
        acc[0] = part_S
        acc[1] = part_a

    @pl.when(r > 0)
    def _accum():
        acc[0] = acc[0] + part_S
        acc[1] = acc[1] + part_a

    @pl.when(r == NB - 1)
    def _final():
        val = (LAM / RHO) * acc[0] / (num_pos * num_neg) + acc[1] / num_pos
        loss_ref[...] = jnp.reshape(val, (1, 1))


def _tc_dense(p_col, yt_col, a_col, i_col, p_row, yt_row, i_row):
    col_spec = lambda dt: pl.BlockSpec((R_BLK, 1), lambda r: (r, 0))
    row_spec = lambda dt: pl.BlockSpec((1, B), lambda r: (0, 0))
    return pl.pallas_call(
        _tc_dense_body,
        grid=(NB,),
        in_specs=[col_spec(None), col_spec(None), col_spec(None), col_spec(None),
                  row_spec(None), row_spec(None), row_spec(None)],
        out_specs=[pl.BlockSpec((R_BLK, 1), lambda r: (r, 0)),
                   pl.BlockSpec((R_BLK, 1), lambda r: (r, 0)),
                   pl.BlockSpec((1, 1), lambda r: (0, 0))],
        out_shape=[jax.ShapeDtypeStruct((B, 1), jnp.float32),
                   jax.ShapeDtypeStruct((B, 1), jnp.float32),
                   jax.ShapeDtypeStruct((1, 1), jnp.float32)],
        scratch_shapes=[pltpu.SMEM((2,), jnp.float32)],
    )(p_col, yt_col, a_col, i_col, p_row, yt_row, i_row)


def _sc_scatter_body(alpha_hbm, idx_hbm, val_hbm, win_hbm, out_hbm,
                     tbl_v, idx_v, val_v, win_v):
    wid = lax.axis_index("s") * NC + lax.axis_index("c")
    lo = wid * CHUNK
    pltpu.sync_copy(alpha_hbm.at[pl.ds(lo, CHUNK)], tbl_v)
    pltpu.sync_copy(idx_hbm, idx_v)
    pltpu.sync_copy(val_hbm, val_v)
    pltpu.sync_copy(win_hbm, win_v)

    def body(k, carry):
        off = pl.multiple_of(k * 16, 16)
        iv = idx_v[pl.ds(off, 16)]
        vv = val_v[pl.ds(off, 16)]
        wv = win_v[pl.ds(off, 16)]
        msk = (iv >= lo) & (iv < lo + CHUNK) & (wv > 0.0)
        plsc.store_scatter(tbl_v, [iv - lo], vv, mask=msk)
        return carry

    lax.fori_loop(0, B // 16, body, 0, unroll=8)
    pltpu.sync_copy(tbl_v, out_hbm.at[pl.ds(lo, CHUNK)])


def _sc_scatter(alpha_pad, index, vals, win):
    mesh = plsc.VectorSubcoreMesh(
        core_axis_name="c", subcore_axis_name="s", num_cores=NC, num_subcores=NS)
    return pl.kernel(
        _sc_scatter_body,
        out_type=jax.ShapeDtypeStruct((PAD_LEN,), jnp.float32),
        mesh=mesh,
        scratch_types=[
            pltpu.VMEM((CHUNK,), jnp.float32),
            pltpu.VMEM((B,), jnp.int32),
            pltpu.VMEM((B,), jnp.float32),
            pltpu.VMEM((B,), jnp.float32),
        ],
        compiler_params=pltpu.CompilerParams(needs_layout_passes=False),
    )(alpha_pad, index, vals, win)


def kernel(y_pred, y_true, index, alpha):
    p_col = y_pred.reshape(B, 1)
    yt_col = y_true.reshape(B, 1)
    index = index.reshape(B).astype(jnp.int32)
    i_col = index.reshape(B, 1)
    alpha_flat = alpha.reshape(DATA_LEN)
    alpha_pad = jnp.pad(alpha_flat, (0, PAD_LEN - DATA_LEN))

    a = _sc_gather(alpha_pad, index)

    slot = jnp.arange(B, dtype=jnp.int32)
    key = index * 8192 + slot
    key_blk = jnp.where(y_true.reshape(B) == 1.0, key, jnp.int32(-(2 ** 30)))
    ps_col = (p_col * SQRT_LOG2E).reshape(B, 1)
    ps_row = ((p_col + MARGIN) * SQRT_LOG2E).reshape(1, B)

    a_new, win, loss = _tc_dense(
        ps_col, yt_col, a.reshape(B, 1), key.reshape(B, 1),
        ps_row, yt_col.reshape(1, B), key_blk.reshape(1, B))

    alpha_out = _sc_scatter(alpha_pad, index, a_new.reshape(B), win.reshape(B))
    return loss.reshape(()), alpha_out[:DATA_LEN].reshape(DATA_LEN, 1)
